# baseline (speedup 1.0000x reference)
import functools

import jax
import jax.numpy as jnp
from jax import lax
from jax.experimental import pallas as pl
from jax.experimental.pallas import tpu as pltpu
from jax.experimental.pallas import tpu_sc as plsc

HIDDEN = 1024
INTER = 2816
E = 8

T = 4096
P = 2 * T
B = 256
NBMAX = P // B + E
NPAD = NBMAX * B
BTR = 512

NW = 32
PPW = P // NW
CH = 64
TPW = T // NW
CC = 32


def _router_body(x_ref, rw_ref, rb_ref, sel_ref, w_ref):
    x = x_ref[...]
    logits = jnp.dot(x, rw_ref[...].T, preferred_element_type=jnp.float32)
    logits = logits + rb_ref[...]
    m = jnp.max(logits, axis=-1, keepdims=True)
    ex = jnp.exp(logits - m)
    probs = ex / jnp.sum(ex, axis=-1, keepdims=True)

    lane = lax.broadcasted_iota(jnp.int32, (BTR, E), 1)
    m1 = jnp.max(probs, axis=-1, keepdims=True)
    a1 = jnp.min(jnp.where(probs == m1, lane, E), axis=-1, keepdims=True)
    probs2 = jnp.where(lane == a1, -1.0, probs)
    m2 = jnp.max(probs2, axis=-1, keepdims=True)
    a2 = jnp.min(jnp.where(probs2 == m2, lane, E), axis=-1, keepdims=True)

    denom = m1 + m2
    w1 = m1 / denom
    w2 = m2 / denom
    zi = jnp.zeros((BTR, 126), jnp.int32)
    zf = jnp.zeros((BTR, 126), jnp.float32)
    sel_ref[...] = jnp.concatenate([a1, a2, zi], axis=-1)
    w_ref[...] = jnp.concatenate([w1, w2, zf], axis=-1)


def _dispatch_body(pairs_ref, pos_ref, eb_ref):
    R = pairs_ref[...]
    r0 = lax.broadcasted_iota(jnp.int32, (128, 128), 0)
    r1 = lax.broadcasted_iota(jnp.int32, (128, 128), 1)
    SU = (r0 < r1).astype(jnp.float32)
    s0 = lax.broadcasted_iota(jnp.int32, (64, 64), 0)
    s1 = lax.broadcasted_iota(jnp.int32, (64, 64), 1)
    SL = (s1 < s0).astype(jnp.float32)

    pos = jnp.zeros((64, 128), jnp.int32)
    blk = lax.broadcasted_iota(jnp.int32, (1, 128), 1)
    ebv = jnp.zeros((1, 128), jnp.int32)
    scal = jnp.zeros((1, 128), jnp.int32)
    base = jnp.int32(0)
    for e in range(E):
        M = (R == e).astype(jnp.float32)
        lanepre = jnp.dot(M, SU, preferred_element_type=jnp.float32)
        tot = jnp.sum(M, axis=1, keepdims=True)
        rowpre = jnp.dot(SL, tot, preferred_element_type=jnp.float32)
        rank = (lanepre + rowpre).astype(jnp.int32)
        cnt = jnp.sum(M).astype(jnp.int32)
        cntpad = ((cnt + B - 1) // B) * B
        pos = jnp.where(R == e, base + rank, pos)
        scal = scal + jnp.where(blk == 40 + e, base, 0)
        scal = scal + jnp.where(blk == 48 + e, cntpad // B, 0)
        base = base + cntpad
        ebv = ebv + (blk * B >= base).astype(jnp.int32)
    pos_ref[...] = pos
    eb_ref[...] = scal + jnp.where(
        blk == 127, base // B,
        jnp.where(blk < 40, jnp.minimum(ebv, E - 1), 0))


IBLK = 256
NI = INTER // IBLK


def _gateup_body(seb_ref, x_any, gw_ref, uw_ref, h_any, x_vmem, h_buf, sem):
    e = pl.program_id(0)
    i = pl.program_id(1)

    @pl.when((e == 0) & (i == 0))
    def _load_x():
        cp = pltpu.make_async_copy(x_any, x_vmem, sem)
        cp.start()
        cp.wait()

    row0 = seb_ref[40 + e]
    nb = seb_ref[48 + e]

    def blkloop(k, carry):
        r = pl.multiple_of(row0 + k * B, B)
        x = x_vmem[pl.ds(r, B), :]
        g = jnp.dot(x, gw_ref[0].T, preferred_element_type=jnp.float32)
        u = jnp.dot(x, uw_ref[0].T, preferred_element_type=jnp.float32)
        h = g * lax.logistic(g) * u
        h_buf[...] = h.astype(jnp.bfloat16)
        cp = pltpu.make_async_copy(
            h_buf, h_any.at[pl.ds(r, B), pl.ds(i * IBLK, IBLK)], sem)
        cp.start()
        cp.wait()
        return carry

    lax.fori_loop(0, nb, blkloop, 0)


def _down_body(seb_ref, h_ref, dw_ref, y_ref):
    b = pl.program_id(0)
    nact = seb_ref[127]

    @pl.when(b < nact)
    def _():
        h = h_ref[...].astype(jnp.float32)
        y_ref[...] = jnp.dot(h, dw_ref[0].T,
                             preferred_element_type=jnp.float32)


def _make_gather():
    mesh = plsc.VectorSubcoreMesh(core_axis_name="c", subcore_axis_name="s")

    @functools.partial(
        pl.kernel, mesh=mesh,
        out_type=jax.ShapeDtypeStruct((NPAD, HIDDEN), jnp.float32),
        scratch_types=[
            pltpu.VMEM((CH,), jnp.int32),
            pltpu.VMEM((CH,), jnp.int32),
            pltpu.VMEM((CH, HIDDEN), jnp.float32),
            pltpu.SemaphoreType.DMA,
        ],
    )
    def gather_k(x_hbm, tok_hbm, pos_hbm, xs_hbm, tok_v, pos_v, rows_v, sem):
        wid = lax.axis_index("s") * 2 + lax.axis_index("c")
        base = wid * PPW

        def chunk(c, carry):
            off = base + c * CH
            pltpu.sync_copy(tok_hbm.at[pl.ds(off, CH)], tok_v)
            pltpu.sync_copy(pos_hbm.at[pl.ds(off, CH)], pos_v)
            pltpu.async_copy(x_hbm.at[tok_v], rows_v, sem).wait()
            pltpu.async_copy(rows_v, xs_hbm.at[pos_v], sem).wait()
            return carry

        lax.fori_loop(0, PPW // CH, chunk, 0)

    return gather_k


def _make_combine():
    mesh = plsc.VectorSubcoreMesh(core_axis_name="c", subcore_axis_name="s")

    @functools.partial(
        pl.kernel, mesh=mesh,
        out_type=jax.ShapeDtypeStruct((T, HIDDEN), jnp.float32),
        scratch_types=[
            pltpu.VMEM((CC,), jnp.int32),
            pltpu.VMEM((CC,), jnp.int32),
            pltpu.VMEM((CC, HIDDEN), jnp.float32),
            pltpu.VMEM((CC, HIDDEN), jnp.float32),
            pltpu.VMEM((CC, 16), jnp.float32),
            pltpu.VMEM((CC, 16), jnp.float32),
            pltpu.VMEM((CC, HIDDEN), jnp.float32),
            pltpu.SemaphoreType.DMA,
        ],
    )
    def combine_k(y_hbm, p0_hbm, p1_hbm, w1_hbm, w2_hbm, out_hbm,
                  i0_v, i1_v, y0_v, y1_v, w1_v, w2_v, o_v, sem):
        wid = lax.axis_index("s") * 2 + lax.axis_index("c")
        base = wid * TPW

        def chunk(c, carry):
            off = base + c * CC
            pltpu.sync_copy(p0_hbm.at[pl.ds(off, CC)], i0_v)
            pltpu.sync_copy(p1_hbm.at[pl.ds(off, CC)], i1_v)
            pltpu.sync_copy(w1_hbm.at[pl.ds(off, CC)], w1_v)
            pltpu.sync_copy(w2_hbm.at[pl.ds(off, CC)], w2_v)
            cp0 = pltpu.async_copy(y_hbm.at[i0_v], y0_v, sem)
            cp1 = pltpu.async_copy(y_hbm.at[i1_v], y1_v, sem)
            cp0.wait()
            cp1.wait()

            def tok(j, carry2):
                wv1 = w1_v[j]
                wv2 = w2_v[j]
                for k in range(HIDDEN // 16):
                    sl = pl.ds(k * 16, 16)
                    o_v[j, sl] = wv1 * y0_v[j, sl] + wv2 * y1_v[j, sl]
                return carry2

            lax.fori_loop(0, CC, tok, 0)
            pltpu.sync_copy(o_v, out_hbm.at[pl.ds(off, CC)])
            return carry

        lax.fori_loop(0, TPW // CC, chunk, 0)

    return combine_k


@jax.jit
def kernel(x, router_w, router_b, gate_w, up_w, down_w):
    batch_shape = x.shape[:-1]
    xf = x.reshape(-1, HIDDEN)

    sel_out, w_out = pl.pallas_call(
        _router_body,
        grid=(T // BTR,),
        in_specs=[
            pl.BlockSpec((BTR, HIDDEN), lambda t: (t, 0)),
            pl.BlockSpec((E, HIDDEN), lambda t: (0, 0)),
            pl.BlockSpec((1, E), lambda t: (0, 0)),
        ],
        out_specs=[
            pl.BlockSpec((BTR, 128), lambda t: (t, 0)),
            pl.BlockSpec((BTR, 128), lambda t: (t, 0)),
        ],
        out_shape=[
            jax.ShapeDtypeStruct((T, 128), jnp.int32),
            jax.ShapeDtypeStruct((T, 128), jnp.float32),
        ],
    )(xf, router_w, router_b.reshape(1, E))

    pairs = sel_out[:, :2].reshape(64, 128)
    pos, eb = pl.pallas_call(
        _dispatch_body,
        in_specs=[pl.BlockSpec((64, 128), lambda: (0, 0))],
        out_specs=[
            pl.BlockSpec((64, 128), lambda: (0, 0)),
            pl.BlockSpec((1, 128), lambda: (0, 0)),
        ],
        out_shape=[
            jax.ShapeDtypeStruct((64, 128), jnp.int32),
            jax.ShapeDtypeStruct((1, 128), jnp.int32),
        ],
    )(pairs)

    pos_flat = pos.reshape(P)
    tok_flat = (jnp.arange(P, dtype=jnp.int32) // 2)
    x_sorted = _make_gather()(xf, tok_flat, pos_flat)

    seb = eb.reshape(128)
    h_sorted = pl.pallas_call(
        _gateup_body,
        grid_spec=pltpu.PrefetchScalarGridSpec(
            num_scalar_prefetch=1,
            grid=(E, NI),
            in_specs=[
                pl.BlockSpec(memory_space=pl.ANY),
                pl.BlockSpec((1, IBLK, HIDDEN),
                             lambda e, i, seb: (e, i, 0)),
                pl.BlockSpec((1, IBLK, HIDDEN),
                             lambda e, i, seb: (e, i, 0)),
            ],
            out_specs=pl.BlockSpec(memory_space=pl.ANY),
            scratch_shapes=[
                pltpu.VMEM((NPAD, HIDDEN), jnp.float32),
                pltpu.VMEM((B, IBLK), jnp.bfloat16),
                pltpu.SemaphoreType.DMA,
            ],
        ),
        out_shape=jax.ShapeDtypeStruct((NPAD, INTER), jnp.bfloat16),
    )(seb, x_sorted, gate_w, up_w)

    y_sorted = pl.pallas_call(
        _down_body,
        grid_spec=pltpu.PrefetchScalarGridSpec(
            num_scalar_prefetch=1,
            grid=(NBMAX,),
            in_specs=[
                pl.BlockSpec((B, INTER), lambda b, seb: (b, 0)),
                pl.BlockSpec((1, HIDDEN, INTER), lambda b, seb: (seb[b], 0, 0)),
            ],
            out_specs=pl.BlockSpec((B, HIDDEN), lambda b, seb: (b, 0)),
        ),
        out_shape=jax.ShapeDtypeStruct((NPAD, HIDDEN), jnp.float32),
    )(seb, h_sorted, down_w)

    p0 = pos_flat[0::2]
    p1 = pos_flat[1::2]
    w1b = jnp.broadcast_to(w_out[:, 0:1], (T, 16))
    w2b = jnp.broadcast_to(w_out[:, 1:2], (T, 16))
    out = _make_combine()(y_sorted, p0, p1, w1b, w2b)

    return out.reshape(*batch_shape, HIDDEN)


# speedup vs baseline: 1.6784x; 1.6784x over previous
import functools

import jax
import jax.numpy as jnp
from jax import lax
from jax.experimental import pallas as pl
from jax.experimental.pallas import tpu as pltpu
from jax.experimental.pallas import tpu_sc as plsc

HIDDEN = 1024
INTER = 2816
E = 8

T = 4096
P = 2 * T
B = 256
NBMAX = P // B + E
NPAD = NBMAX * B
BTR = 512

NW = 32
PPW = P // NW
CH = 64
TPW = T // NW
CC = 32


def _router_body(x_ref, rw_ref, rb_ref, sel_ref, w_ref):
    x = x_ref[...]
    logits = jnp.dot(x, rw_ref[...].T, preferred_element_type=jnp.float32)
    logits = logits + rb_ref[...]
    m = jnp.max(logits, axis=-1, keepdims=True)
    ex = jnp.exp(logits - m)
    probs = ex / jnp.sum(ex, axis=-1, keepdims=True)

    lane = lax.broadcasted_iota(jnp.int32, (BTR, E), 1)
    m1 = jnp.max(probs, axis=-1, keepdims=True)
    a1 = jnp.min(jnp.where(probs == m1, lane, E), axis=-1, keepdims=True)
    probs2 = jnp.where(lane == a1, -1.0, probs)
    m2 = jnp.max(probs2, axis=-1, keepdims=True)
    a2 = jnp.min(jnp.where(probs2 == m2, lane, E), axis=-1, keepdims=True)

    denom = m1 + m2
    w1 = m1 / denom
    w2 = m2 / denom
    zi = jnp.zeros((BTR, 126), jnp.int32)
    zf = jnp.zeros((BTR, 126), jnp.float32)
    sel_ref[...] = jnp.concatenate([a1, a2, zi], axis=-1)
    w_ref[...] = jnp.concatenate([w1, w2, zf], axis=-1)


def _dispatch_body(pairs_ref, pos_ref, eb_ref):
    R = pairs_ref[...]
    r0 = lax.broadcasted_iota(jnp.int32, (128, 128), 0)
    r1 = lax.broadcasted_iota(jnp.int32, (128, 128), 1)
    SU = (r0 < r1).astype(jnp.float32)
    s0 = lax.broadcasted_iota(jnp.int32, (64, 64), 0)
    s1 = lax.broadcasted_iota(jnp.int32, (64, 64), 1)
    SL = (s1 < s0).astype(jnp.float32)

    pos = jnp.zeros((64, 128), jnp.int32)
    blk = lax.broadcasted_iota(jnp.int32, (1, 128), 1)
    ebv = jnp.zeros((1, 128), jnp.int32)
    base = jnp.int32(0)
    for e in range(E):
        M = (R == e).astype(jnp.float32)
        lanepre = jnp.dot(M, SU, preferred_element_type=jnp.float32)
        tot = jnp.sum(M, axis=1, keepdims=True)
        rowpre = jnp.dot(SL, tot, preferred_element_type=jnp.float32)
        rank = (lanepre + rowpre).astype(jnp.int32)
        cnt = jnp.sum(M).astype(jnp.int32)
        cntpad = ((cnt + B - 1) // B) * B
        pos = jnp.where(R == e, base + rank, pos)
        base = base + cntpad
        ebv = ebv + (blk * B >= base).astype(jnp.int32)
    pos_ref[...] = pos
    eb_ref[...] = jnp.where(blk == 127, base // B, jnp.minimum(ebv, E - 1))


IBLK = 1408
NI = INTER // IBLK


def _gateup_body(seb_ref, x_ref, gw_ref, uw_ref, h_ref):
    b = pl.program_id(1)
    nact = seb_ref[127]

    @pl.when(b < nact)
    def _():
        x = x_ref[...]
        g = jnp.dot(x, gw_ref[0].T, preferred_element_type=jnp.float32)
        u = jnp.dot(x, uw_ref[0].T, preferred_element_type=jnp.float32)
        h = g * lax.logistic(g) * u
        h_ref[...] = h.astype(jnp.bfloat16)


def _down_body(seb_ref, h_ref, dw_ref, y_ref):
    b = pl.program_id(0)
    nact = seb_ref[127]

    @pl.when(b < nact)
    def _():
        h = h_ref[...].astype(jnp.float32)
        y_ref[...] = jnp.dot(h, dw_ref[0].T,
                             preferred_element_type=jnp.float32)


def _make_gather():
    mesh = plsc.VectorSubcoreMesh(core_axis_name="c", subcore_axis_name="s")

    @functools.partial(
        pl.kernel, mesh=mesh,
        out_type=jax.ShapeDtypeStruct((NPAD, HIDDEN), jnp.float32),
        scratch_types=[
            pltpu.VMEM((CH,), jnp.int32),
            pltpu.VMEM((CH,), jnp.int32),
            pltpu.VMEM((CH, HIDDEN), jnp.float32),
            pltpu.SemaphoreType.DMA,
        ],
    )
    def gather_k(x_hbm, tok_hbm, pos_hbm, xs_hbm, tok_v, pos_v, rows_v, sem):
        wid = lax.axis_index("s") * 2 + lax.axis_index("c")
        base = wid * PPW

        def chunk(c, carry):
            off = base + c * CH
            pltpu.sync_copy(tok_hbm.at[pl.ds(off, CH)], tok_v)
            pltpu.sync_copy(pos_hbm.at[pl.ds(off, CH)], pos_v)
            pltpu.async_copy(x_hbm.at[tok_v], rows_v, sem).wait()
            pltpu.async_copy(rows_v, xs_hbm.at[pos_v], sem).wait()
            return carry

        lax.fori_loop(0, PPW // CH, chunk, 0)

    return gather_k


def _make_combine():
    mesh = plsc.VectorSubcoreMesh(core_axis_name="c", subcore_axis_name="s")

    @functools.partial(
        pl.kernel, mesh=mesh,
        out_type=jax.ShapeDtypeStruct((T, HIDDEN), jnp.float32),
        scratch_types=[
            pltpu.VMEM((CC,), jnp.int32),
            pltpu.VMEM((CC,), jnp.int32),
            pltpu.VMEM((CC, HIDDEN), jnp.float32),
            pltpu.VMEM((CC, HIDDEN), jnp.float32),
            pltpu.VMEM((CC, 16), jnp.float32),
            pltpu.VMEM((CC, 16), jnp.float32),
            pltpu.VMEM((CC, HIDDEN), jnp.float32),
            pltpu.SemaphoreType.DMA,
        ],
    )
    def combine_k(y_hbm, p0_hbm, p1_hbm, w1_hbm, w2_hbm, out_hbm,
                  i0_v, i1_v, y0_v, y1_v, w1_v, w2_v, o_v, sem):
        wid = lax.axis_index("s") * 2 + lax.axis_index("c")
        base = wid * TPW

        def chunk(c, carry):
            off = base + c * CC
            pltpu.sync_copy(p0_hbm.at[pl.ds(off, CC)], i0_v)
            pltpu.sync_copy(p1_hbm.at[pl.ds(off, CC)], i1_v)
            pltpu.sync_copy(w1_hbm.at[pl.ds(off, CC)], w1_v)
            pltpu.sync_copy(w2_hbm.at[pl.ds(off, CC)], w2_v)
            cp0 = pltpu.async_copy(y_hbm.at[i0_v], y0_v, sem)
            cp1 = pltpu.async_copy(y_hbm.at[i1_v], y1_v, sem)
            cp0.wait()
            cp1.wait()

            def tok(j, carry2):
                wv1 = w1_v[j]
                wv2 = w2_v[j]
                for k in range(HIDDEN // 16):
                    sl = pl.ds(k * 16, 16)
                    o_v[j, sl] = wv1 * y0_v[j, sl] + wv2 * y1_v[j, sl]
                return carry2

            lax.fori_loop(0, CC, tok, 0)
            pltpu.sync_copy(o_v, out_hbm.at[pl.ds(off, CC)])
            return carry

        lax.fori_loop(0, TPW // CC, chunk, 0)

    return combine_k


@jax.jit
def kernel(x, router_w, router_b, gate_w, up_w, down_w):
    batch_shape = x.shape[:-1]
    xf = x.reshape(-1, HIDDEN)

    sel_out, w_out = pl.pallas_call(
        _router_body,
        grid=(T // BTR,),
        in_specs=[
            pl.BlockSpec((BTR, HIDDEN), lambda t: (t, 0)),
            pl.BlockSpec((E, HIDDEN), lambda t: (0, 0)),
            pl.BlockSpec((1, E), lambda t: (0, 0)),
        ],
        out_specs=[
            pl.BlockSpec((BTR, 128), lambda t: (t, 0)),
            pl.BlockSpec((BTR, 128), lambda t: (t, 0)),
        ],
        out_shape=[
            jax.ShapeDtypeStruct((T, 128), jnp.int32),
            jax.ShapeDtypeStruct((T, 128), jnp.float32),
        ],
    )(xf, router_w, router_b.reshape(1, E))

    pairs = sel_out[:, :2].reshape(64, 128)
    pos, eb = pl.pallas_call(
        _dispatch_body,
        in_specs=[pl.BlockSpec((64, 128), lambda: (0, 0))],
        out_specs=[
            pl.BlockSpec((64, 128), lambda: (0, 0)),
            pl.BlockSpec((1, 128), lambda: (0, 0)),
        ],
        out_shape=[
            jax.ShapeDtypeStruct((64, 128), jnp.int32),
            jax.ShapeDtypeStruct((1, 128), jnp.int32),
        ],
    )(pairs)

    pos_flat = pos.reshape(P)
    tok_flat = (jnp.arange(P, dtype=jnp.int32) // 2)
    x_sorted = _make_gather()(xf, tok_flat, pos_flat)

    seb = eb.reshape(128)
    h_sorted = pl.pallas_call(
        _gateup_body,
        grid_spec=pltpu.PrefetchScalarGridSpec(
            num_scalar_prefetch=1,
            grid=(NI, NBMAX),
            in_specs=[
                pl.BlockSpec((B, HIDDEN), lambda i, b, seb: (b, 0)),
                pl.BlockSpec((1, IBLK, HIDDEN),
                             lambda i, b, seb: (seb[b], i, 0)),
                pl.BlockSpec((1, IBLK, HIDDEN),
                             lambda i, b, seb: (seb[b], i, 0)),
            ],
            out_specs=pl.BlockSpec((B, IBLK), lambda i, b, seb: (b, i)),
        ),
        out_shape=jax.ShapeDtypeStruct((NPAD, INTER), jnp.bfloat16),
    )(seb, x_sorted, gate_w, up_w)

    y_sorted = pl.pallas_call(
        _down_body,
        grid_spec=pltpu.PrefetchScalarGridSpec(
            num_scalar_prefetch=1,
            grid=(NBMAX,),
            in_specs=[
                pl.BlockSpec((B, INTER), lambda b, seb: (b, 0)),
                pl.BlockSpec((1, HIDDEN, INTER), lambda b, seb: (seb[b], 0, 0)),
            ],
            out_specs=pl.BlockSpec((B, HIDDEN), lambda b, seb: (b, 0)),
        ),
        out_shape=jax.ShapeDtypeStruct((NPAD, HIDDEN), jnp.float32),
    )(seb, h_sorted, down_w)

    p0 = pos_flat[0::2]
    p1 = pos_flat[1::2]
    w1b = jnp.broadcast_to(w_out[:, 0:1], (T, 16))
    w2b = jnp.broadcast_to(w_out[:, 1:2], (T, 16))
    out = _make_combine()(y_sorted, p0, p1, w1b, w2b)

    return out.reshape(*batch_shape, HIDDEN)


# speedup vs baseline: 1.7395x; 1.0364x over previous
import functools

import jax
import jax.numpy as jnp
from jax import lax
from jax.experimental import pallas as pl
from jax.experimental.pallas import tpu as pltpu
from jax.experimental.pallas import tpu_sc as plsc

HIDDEN = 1024
INTER = 2816
E = 8

T = 4096
P = 2 * T
B = 256
NBMAX = P // B + E
NPAD = NBMAX * B
BTR = 512

NW = 32
PPW = P // NW
CH = 32
NCH = PPW // CH
TPW = T // NW
CC = 16
NCC = TPW // CC


def _router_body(x_ref, rw_ref, rb_ref, sel_ref, w_ref):
    x = x_ref[...]
    logits = jnp.dot(x, rw_ref[...].T, preferred_element_type=jnp.float32)
    logits = logits + rb_ref[...]
    m = jnp.max(logits, axis=-1, keepdims=True)
    ex = jnp.exp(logits - m)
    probs = ex / jnp.sum(ex, axis=-1, keepdims=True)

    lane = lax.broadcasted_iota(jnp.int32, (BTR, E), 1)
    m1 = jnp.max(probs, axis=-1, keepdims=True)
    a1 = jnp.min(jnp.where(probs == m1, lane, E), axis=-1, keepdims=True)
    probs2 = jnp.where(lane == a1, -1.0, probs)
    m2 = jnp.max(probs2, axis=-1, keepdims=True)
    a2 = jnp.min(jnp.where(probs2 == m2, lane, E), axis=-1, keepdims=True)

    denom = m1 + m2
    w1 = m1 / denom
    w2 = m2 / denom
    zi = jnp.zeros((BTR, 126), jnp.int32)
    zf = jnp.zeros((BTR, 126), jnp.float32)
    sel_ref[...] = jnp.concatenate([a1, a2, zi], axis=-1)
    w_ref[...] = jnp.concatenate([w1, w2, zf], axis=-1)


def _dispatch_body(pairs_ref, pos_ref, eb_ref):
    R = pairs_ref[...]
    r0 = lax.broadcasted_iota(jnp.int32, (128, 128), 0)
    r1 = lax.broadcasted_iota(jnp.int32, (128, 128), 1)
    SU = (r0 < r1).astype(jnp.float32)
    s0 = lax.broadcasted_iota(jnp.int32, (64, 64), 0)
    s1 = lax.broadcasted_iota(jnp.int32, (64, 64), 1)
    SL = (s1 < s0).astype(jnp.float32)

    pos = jnp.zeros((64, 128), jnp.int32)
    blk = lax.broadcasted_iota(jnp.int32, (1, 128), 1)
    ebv = jnp.zeros((1, 128), jnp.int32)
    base = jnp.int32(0)
    for e in range(E):
        M = (R == e).astype(jnp.float32)
        lanepre = jnp.dot(M, SU, preferred_element_type=jnp.float32)
        tot = jnp.sum(M, axis=1, keepdims=True)
        rowpre = jnp.dot(SL, tot, preferred_element_type=jnp.float32)
        rank = (lanepre + rowpre).astype(jnp.int32)
        cnt = jnp.sum(M).astype(jnp.int32)
        cntpad = ((cnt + B - 1) // B) * B
        pos = jnp.where(R == e, base + rank, pos)
        base = base + cntpad
        ebv = ebv + (blk * B >= base).astype(jnp.int32)
    pos_ref[...] = pos
    eb_ref[...] = jnp.where(blk == 127, base // B, jnp.minimum(ebv, E - 1))


IBLK = 1408
NI = INTER // IBLK


def _gateup_body(seb_ref, x_ref, gw_ref, uw_ref, h_ref):
    b = pl.program_id(1)
    nact = seb_ref[127]

    @pl.when(b < nact)
    def _():
        x = x_ref[...]
        g = jnp.dot(x, gw_ref[0].T, preferred_element_type=jnp.float32)
        u = jnp.dot(x, uw_ref[0].T, preferred_element_type=jnp.float32)
        h = g * lax.logistic(g) * u
        h_ref[...] = h.astype(jnp.bfloat16)


def _down_body(seb_ref, h_ref, dw_ref, y_ref):
    b = pl.program_id(0)
    nact = seb_ref[127]

    @pl.when(b < nact)
    def _():
        h = h_ref[...].astype(jnp.float32)
        y_ref[...] = jnp.dot(h, dw_ref[0].T,
                             preferred_element_type=jnp.float32)


def _make_gather():
    mesh = plsc.VectorSubcoreMesh(core_axis_name="c", subcore_axis_name="s")

    @functools.partial(
        pl.kernel, mesh=mesh,
        out_type=jax.ShapeDtypeStruct((NPAD, HIDDEN), jnp.float32),
        scratch_types=[
            pltpu.VMEM((PPW,), jnp.int32),
            pltpu.VMEM((PPW,), jnp.int32),
            pltpu.VMEM((CH, HIDDEN), jnp.float32),
            pltpu.VMEM((CH, HIDDEN), jnp.float32),
            pltpu.SemaphoreType.DMA,
            pltpu.SemaphoreType.DMA,
            pltpu.SemaphoreType.DMA,
            pltpu.SemaphoreType.DMA,
        ],
    )
    def gather_k(x_hbm, tok_hbm, pos_hbm, xs_hbm, tok_v, pos_v,
                 rows0, rows1, sg0, sg1, ss0, ss1):
        wid = lax.axis_index("s") * 2 + lax.axis_index("c")
        base = wid * PPW
        pltpu.sync_copy(tok_hbm.at[pl.ds(base, PPW)], tok_v)
        pltpu.sync_copy(pos_hbm.at[pl.ds(base, PPW)], pos_v)
        rows = (rows0, rows1)
        sg = (sg0, sg1)
        ss = (ss0, ss1)

        def g_start(c, s):
            idx = tok_v.at[pl.ds(c * CH, CH)]
            return pltpu.async_copy(x_hbm.at[idx], rows[s], sg[s])

        def s_start(c, s):
            idx = pos_v.at[pl.ds(c * CH, CH)]
            return pltpu.async_copy(rows[s], xs_hbm.at[idx], ss[s])

        gath = [None, None]
        scat = [None, None]
        gath[0] = g_start(0, 0)
        for c in range(NCH):
            s = c % 2
            if c + 1 < NCH:
                if scat[1 - s] is not None:
                    scat[1 - s].wait()
                    scat[1 - s] = None
                gath[1 - s] = g_start(c + 1, 1 - s)
            gath[s].wait()
            scat[s] = s_start(c, s)
        for s in range(2):
            if scat[s] is not None:
                scat[s].wait()

    return gather_k


def _make_combine():
    mesh = plsc.VectorSubcoreMesh(core_axis_name="c", subcore_axis_name="s")

    @functools.partial(
        pl.kernel, mesh=mesh,
        out_type=jax.ShapeDtypeStruct((T, HIDDEN), jnp.float32),
        scratch_types=[
            pltpu.VMEM((TPW,), jnp.int32),
            pltpu.VMEM((TPW,), jnp.int32),
            pltpu.VMEM((TPW, 16), jnp.float32),
            pltpu.VMEM((TPW, 16), jnp.float32),
            pltpu.VMEM((CC, HIDDEN), jnp.float32),
            pltpu.VMEM((CC, HIDDEN), jnp.float32),
            pltpu.VMEM((CC, HIDDEN), jnp.float32),
            pltpu.VMEM((CC, HIDDEN), jnp.float32),
            pltpu.SemaphoreType.DMA,
            pltpu.SemaphoreType.DMA,
            pltpu.SemaphoreType.DMA,
            pltpu.SemaphoreType.DMA,
            pltpu.SemaphoreType.DMA,
            pltpu.SemaphoreType.DMA,
        ],
    )
    def combine_k(y_hbm, p0_hbm, p1_hbm, w1_hbm, w2_hbm, out_hbm,
                  i0_v, i1_v, w1_v, w2_v, y0a, y0b, y1a, y1b,
                  sa0, sb0, sa1, sb1, soa, sob):
        wid = lax.axis_index("s") * 2 + lax.axis_index("c")
        base = wid * TPW
        pltpu.sync_copy(p0_hbm.at[pl.ds(base, TPW)], i0_v)
        pltpu.sync_copy(p1_hbm.at[pl.ds(base, TPW)], i1_v)
        pltpu.sync_copy(w1_hbm.at[pl.ds(base, TPW)], w1_v)
        pltpu.sync_copy(w2_hbm.at[pl.ds(base, TPW)], w2_v)
        y0 = (y0a, y0b)
        y1 = (y1a, y1b)
        s0 = (sa0, sb0)
        s1 = (sa1, sb1)
        so = (soa, sob)

        def g_start(c, s):
            cp0 = pltpu.async_copy(
                y_hbm.at[i0_v.at[pl.ds(c * CC, CC)]], y0[s], s0[s])
            cp1 = pltpu.async_copy(
                y_hbm.at[i1_v.at[pl.ds(c * CC, CC)]], y1[s], s1[s])
            return cp0, cp1

        gath = [None, None]
        stor = [None, None]
        gath[0] = g_start(0, 0)
        for c in range(NCC):
            s = c % 2
            if c + 1 < NCC:
                if stor[1 - s] is not None:
                    stor[1 - s].wait()
                    stor[1 - s] = None
                gath[1 - s] = g_start(c + 1, 1 - s)
            gath[s][0].wait()
            gath[s][1].wait()

            def tok(j, carry2, _s=s, _c=c):
                wv1 = w1_v[_c * CC + j]
                wv2 = w2_v[_c * CC + j]
                for k in range(HIDDEN // 16):
                    sl = pl.ds(k * 16, 16)
                    y0[_s][j, sl] = wv1 * y0[_s][j, sl] + wv2 * y1[_s][j, sl]
                return carry2

            lax.fori_loop(0, CC, tok, 0)
            stor[s] = pltpu.async_copy(
                y0[s], out_hbm.at[pl.ds(base + c * CC, CC)], so[s])
        for s in range(2):
            if stor[s] is not None:
                stor[s].wait()

    return combine_k


@jax.jit
def kernel(x, router_w, router_b, gate_w, up_w, down_w):
    batch_shape = x.shape[:-1]
    xf = x.reshape(-1, HIDDEN)

    sel_out, w_out = pl.pallas_call(
        _router_body,
        grid=(T // BTR,),
        in_specs=[
            pl.BlockSpec((BTR, HIDDEN), lambda t: (t, 0)),
            pl.BlockSpec((E, HIDDEN), lambda t: (0, 0)),
            pl.BlockSpec((1, E), lambda t: (0, 0)),
        ],
        out_specs=[
            pl.BlockSpec((BTR, 128), lambda t: (t, 0)),
            pl.BlockSpec((BTR, 128), lambda t: (t, 0)),
        ],
        out_shape=[
            jax.ShapeDtypeStruct((T, 128), jnp.int32),
            jax.ShapeDtypeStruct((T, 128), jnp.float32),
        ],
    )(xf, router_w, router_b.reshape(1, E))

    pairs = sel_out[:, :2].reshape(64, 128)
    pos, eb = pl.pallas_call(
        _dispatch_body,
        in_specs=[pl.BlockSpec((64, 128), lambda: (0, 0))],
        out_specs=[
            pl.BlockSpec((64, 128), lambda: (0, 0)),
            pl.BlockSpec((1, 128), lambda: (0, 0)),
        ],
        out_shape=[
            jax.ShapeDtypeStruct((64, 128), jnp.int32),
            jax.ShapeDtypeStruct((1, 128), jnp.int32),
        ],
    )(pairs)

    pos_flat = pos.reshape(P)
    tok_flat = (jnp.arange(P, dtype=jnp.int32) // 2)
    x_sorted = _make_gather()(xf, tok_flat, pos_flat)

    seb = eb.reshape(128)
    h_sorted = pl.pallas_call(
        _gateup_body,
        grid_spec=pltpu.PrefetchScalarGridSpec(
            num_scalar_prefetch=1,
            grid=(NI, NBMAX),
            in_specs=[
                pl.BlockSpec((B, HIDDEN), lambda i, b, seb: (b, 0)),
                pl.BlockSpec((1, IBLK, HIDDEN),
                             lambda i, b, seb: (seb[b], i, 0)),
                pl.BlockSpec((1, IBLK, HIDDEN),
                             lambda i, b, seb: (seb[b], i, 0)),
            ],
            out_specs=pl.BlockSpec((B, IBLK), lambda i, b, seb: (b, i)),
        ),
        out_shape=jax.ShapeDtypeStruct((NPAD, INTER), jnp.bfloat16),
    )(seb, x_sorted, gate_w, up_w)

    y_sorted = pl.pallas_call(
        _down_body,
        grid_spec=pltpu.PrefetchScalarGridSpec(
            num_scalar_prefetch=1,
            grid=(NBMAX,),
            in_specs=[
                pl.BlockSpec((B, INTER), lambda b, seb: (b, 0)),
                pl.BlockSpec((1, HIDDEN, INTER), lambda b, seb: (seb[b], 0, 0)),
            ],
            out_specs=pl.BlockSpec((B, HIDDEN), lambda b, seb: (b, 0)),
        ),
        out_shape=jax.ShapeDtypeStruct((NPAD, HIDDEN), jnp.float32),
    )(seb, h_sorted, down_w)

    p0 = pos_flat[0::2]
    p1 = pos_flat[1::2]
    w1b = jnp.broadcast_to(w_out[:, 0:1], (T, 16))
    w2b = jnp.broadcast_to(w_out[:, 1:2], (T, 16))
    out = _make_combine()(y_sorted, p0, p1, w1b, w2b)

    return out.reshape(*batch_shape, HIDDEN)


# speedup vs baseline: 1.9002x; 1.0924x over previous
import functools

import jax
import jax.numpy as jnp
from jax import lax
from jax.experimental import pallas as pl
from jax.experimental.pallas import tpu as pltpu
from jax.experimental.pallas import tpu_sc as plsc

HIDDEN = 1024
INTER = 2816
E = 8

T = 4096
P = 2 * T
B = 256
NBMAX = P // B + E
NPAD = NBMAX * B
BTR = 512

NW = 32
PPW = P // NW
CH = 32
NCH = PPW // CH
TPW = T // NW
CC = 16
NCC = TPW // CC


def _router_body(x_ref, rw_ref, rb_ref, sel_ref, w_ref):
    x = x_ref[...]
    logits = jnp.dot(x, rw_ref[...].T, preferred_element_type=jnp.float32)
    logits = logits + rb_ref[...]
    m = jnp.max(logits, axis=-1, keepdims=True)
    ex = jnp.exp(logits - m)
    probs = ex / jnp.sum(ex, axis=-1, keepdims=True)

    lane = lax.broadcasted_iota(jnp.int32, (BTR, E), 1)
    m1 = jnp.max(probs, axis=-1, keepdims=True)
    a1 = jnp.min(jnp.where(probs == m1, lane, E), axis=-1, keepdims=True)
    probs2 = jnp.where(lane == a1, -1.0, probs)
    m2 = jnp.max(probs2, axis=-1, keepdims=True)
    a2 = jnp.min(jnp.where(probs2 == m2, lane, E), axis=-1, keepdims=True)

    denom = m1 + m2
    w1 = m1 / denom
    w2 = m2 / denom
    zi = jnp.zeros((BTR, 126), jnp.int32)
    zf = jnp.zeros((BTR, 126), jnp.float32)
    sel_ref[...] = jnp.concatenate([a1, a2, zi], axis=-1)
    w_ref[...] = jnp.concatenate([w1, w2, zf], axis=-1)


def _dispatch_body(pairs_ref, pos_ref, eb_ref):
    R = pairs_ref[...]
    r0 = lax.broadcasted_iota(jnp.int32, (128, 128), 0)
    r1 = lax.broadcasted_iota(jnp.int32, (128, 128), 1)
    SU = (r0 < r1).astype(jnp.float32)
    s0 = lax.broadcasted_iota(jnp.int32, (64, 64), 0)
    s1 = lax.broadcasted_iota(jnp.int32, (64, 64), 1)
    SL = (s1 < s0).astype(jnp.float32)

    pos = jnp.zeros((64, 128), jnp.int32)
    blk = lax.broadcasted_iota(jnp.int32, (1, 128), 1)
    ebv = jnp.zeros((1, 128), jnp.int32)
    base = jnp.int32(0)
    for e in range(E):
        M = (R == e).astype(jnp.float32)
        lanepre = jnp.dot(M, SU, preferred_element_type=jnp.float32)
        tot = jnp.sum(M, axis=1, keepdims=True)
        rowpre = jnp.dot(SL, tot, preferred_element_type=jnp.float32)
        rank = (lanepre + rowpre).astype(jnp.int32)
        cnt = jnp.sum(M).astype(jnp.int32)
        cntpad = ((cnt + B - 1) // B) * B
        pos = jnp.where(R == e, base + rank, pos)
        base = base + cntpad
        ebv = ebv + (blk * B >= base).astype(jnp.int32)
    pos_ref[...] = pos
    eb_ref[...] = jnp.where(blk == 127, base // B, jnp.minimum(ebv, E - 1))


IBLK = 2816
NI = INTER // IBLK


def _gateup_body(seb_ref, x_ref, gw_ref, uw_ref, h_ref):
    b = pl.program_id(1)
    nact = seb_ref[127]

    @pl.when(b < nact)
    def _():
        x = x_ref[...]
        g = jnp.dot(x, gw_ref[0].T, preferred_element_type=jnp.float32)
        u = jnp.dot(x, uw_ref[0].T, preferred_element_type=jnp.float32)
        h = g * lax.logistic(g) * u
        h_ref[...] = h.astype(jnp.bfloat16)


def _down_body(seb_ref, h_ref, dw_ref, y_ref):
    b = pl.program_id(0)
    nact = seb_ref[127]

    @pl.when(b < nact)
    def _():
        h = h_ref[...].astype(jnp.float32)
        y_ref[...] = jnp.dot(h, dw_ref[0].T,
                             preferred_element_type=jnp.float32)


def _make_gather():
    mesh = plsc.VectorSubcoreMesh(core_axis_name="c", subcore_axis_name="s")

    @functools.partial(
        pl.kernel, mesh=mesh,
        out_type=jax.ShapeDtypeStruct((NPAD, HIDDEN), jnp.float32),
        scratch_types=[
            pltpu.VMEM((PPW,), jnp.int32),
            pltpu.VMEM((PPW,), jnp.int32),
            pltpu.VMEM((CH, HIDDEN), jnp.float32),
            pltpu.VMEM((CH, HIDDEN), jnp.float32),
            pltpu.SemaphoreType.DMA,
            pltpu.SemaphoreType.DMA,
            pltpu.SemaphoreType.DMA,
            pltpu.SemaphoreType.DMA,
        ],
    )
    def gather_k(x_hbm, tok_hbm, pos_hbm, xs_hbm, tok_v, pos_v,
                 rows0, rows1, sg0, sg1, ss0, ss1):
        wid = lax.axis_index("s") * 2 + lax.axis_index("c")
        base = wid * PPW
        pltpu.sync_copy(tok_hbm.at[pl.ds(base, PPW)], tok_v)
        pltpu.sync_copy(pos_hbm.at[pl.ds(base, PPW)], pos_v)
        rows = (rows0, rows1)
        sg = (sg0, sg1)
        ss = (ss0, ss1)

        def g_start(c, s):
            idx = tok_v.at[pl.ds(c * CH, CH)]
            return pltpu.async_copy(x_hbm.at[idx], rows[s], sg[s])

        def s_start(c, s):
            idx = pos_v.at[pl.ds(c * CH, CH)]
            return pltpu.async_copy(rows[s], xs_hbm.at[idx], ss[s])

        gath = [None, None]
        scat = [None, None]
        gath[0] = g_start(0, 0)
        for c in range(NCH):
            s = c % 2
            if c + 1 < NCH:
                if scat[1 - s] is not None:
                    scat[1 - s].wait()
                    scat[1 - s] = None
                gath[1 - s] = g_start(c + 1, 1 - s)
            gath[s].wait()
            scat[s] = s_start(c, s)
        for s in range(2):
            if scat[s] is not None:
                scat[s].wait()

    return gather_k


def _make_combine():
    mesh = plsc.VectorSubcoreMesh(core_axis_name="c", subcore_axis_name="s")

    @functools.partial(
        pl.kernel, mesh=mesh,
        out_type=jax.ShapeDtypeStruct((T, HIDDEN), jnp.float32),
        scratch_types=[
            pltpu.VMEM((TPW,), jnp.int32),
            pltpu.VMEM((TPW,), jnp.int32),
            pltpu.VMEM((TPW, 16), jnp.float32),
            pltpu.VMEM((TPW, 16), jnp.float32),
            pltpu.VMEM((CC, HIDDEN), jnp.float32),
            pltpu.VMEM((CC, HIDDEN), jnp.float32),
            pltpu.VMEM((CC, HIDDEN), jnp.float32),
            pltpu.VMEM((CC, HIDDEN), jnp.float32),
            pltpu.SemaphoreType.DMA,
            pltpu.SemaphoreType.DMA,
            pltpu.SemaphoreType.DMA,
            pltpu.SemaphoreType.DMA,
            pltpu.SemaphoreType.DMA,
            pltpu.SemaphoreType.DMA,
        ],
    )
    def combine_k(y_hbm, p0_hbm, p1_hbm, w1_hbm, w2_hbm, out_hbm,
                  i0_v, i1_v, w1_v, w2_v, y0a, y0b, y1a, y1b,
                  sa0, sb0, sa1, sb1, soa, sob):
        wid = lax.axis_index("s") * 2 + lax.axis_index("c")
        base = wid * TPW
        pltpu.sync_copy(p0_hbm.at[pl.ds(base, TPW)], i0_v)
        pltpu.sync_copy(p1_hbm.at[pl.ds(base, TPW)], i1_v)
        pltpu.sync_copy(w1_hbm.at[pl.ds(base, TPW)], w1_v)
        pltpu.sync_copy(w2_hbm.at[pl.ds(base, TPW)], w2_v)
        y0 = (y0a, y0b)
        y1 = (y1a, y1b)
        s0 = (sa0, sb0)
        s1 = (sa1, sb1)
        so = (soa, sob)

        def g_start(c, s):
            cp0 = pltpu.async_copy(
                y_hbm.at[i0_v.at[pl.ds(c * CC, CC)]], y0[s], s0[s])
            cp1 = pltpu.async_copy(
                y_hbm.at[i1_v.at[pl.ds(c * CC, CC)]], y1[s], s1[s])
            return cp0, cp1

        gath = [None, None]
        stor = [None, None]
        gath[0] = g_start(0, 0)
        for c in range(NCC):
            s = c % 2
            if c + 1 < NCC:
                if stor[1 - s] is not None:
                    stor[1 - s].wait()
                    stor[1 - s] = None
                gath[1 - s] = g_start(c + 1, 1 - s)
            gath[s][0].wait()
            gath[s][1].wait()

            def tok(j, carry2, _s=s, _c=c):
                wv1 = w1_v[_c * CC + j]
                wv2 = w2_v[_c * CC + j]
                for k in range(HIDDEN // 16):
                    sl = pl.ds(k * 16, 16)
                    y0[_s][j, sl] = wv1 * y0[_s][j, sl] + wv2 * y1[_s][j, sl]
                return carry2

            lax.fori_loop(0, CC, tok, 0)
            stor[s] = pltpu.async_copy(
                y0[s], out_hbm.at[pl.ds(base + c * CC, CC)], so[s])
        for s in range(2):
            if stor[s] is not None:
                stor[s].wait()

    return combine_k


@jax.jit
def kernel(x, router_w, router_b, gate_w, up_w, down_w):
    batch_shape = x.shape[:-1]
    xf = x.reshape(-1, HIDDEN)

    sel_out, w_out = pl.pallas_call(
        _router_body,
        grid=(T // BTR,),
        in_specs=[
            pl.BlockSpec((BTR, HIDDEN), lambda t: (t, 0)),
            pl.BlockSpec((E, HIDDEN), lambda t: (0, 0)),
            pl.BlockSpec((1, E), lambda t: (0, 0)),
        ],
        out_specs=[
            pl.BlockSpec((BTR, 128), lambda t: (t, 0)),
            pl.BlockSpec((BTR, 128), lambda t: (t, 0)),
        ],
        out_shape=[
            jax.ShapeDtypeStruct((T, 128), jnp.int32),
            jax.ShapeDtypeStruct((T, 128), jnp.float32),
        ],
    )(xf, router_w, router_b.reshape(1, E))

    pairs = sel_out[:, :2].reshape(64, 128)
    pos, eb = pl.pallas_call(
        _dispatch_body,
        in_specs=[pl.BlockSpec((64, 128), lambda: (0, 0))],
        out_specs=[
            pl.BlockSpec((64, 128), lambda: (0, 0)),
            pl.BlockSpec((1, 128), lambda: (0, 0)),
        ],
        out_shape=[
            jax.ShapeDtypeStruct((64, 128), jnp.int32),
            jax.ShapeDtypeStruct((1, 128), jnp.int32),
        ],
    )(pairs)

    pos_flat = pos.reshape(P)
    tok_flat = (jnp.arange(P, dtype=jnp.int32) // 2)
    x_sorted = _make_gather()(xf, tok_flat, pos_flat)

    seb = eb.reshape(128)
    h_sorted = pl.pallas_call(
        _gateup_body,
        grid_spec=pltpu.PrefetchScalarGridSpec(
            num_scalar_prefetch=1,
            grid=(NI, NBMAX),
            in_specs=[
                pl.BlockSpec((B, HIDDEN), lambda i, b, seb: (b, 0)),
                pl.BlockSpec((1, IBLK, HIDDEN),
                             lambda i, b, seb: (seb[b], i, 0)),
                pl.BlockSpec((1, IBLK, HIDDEN),
                             lambda i, b, seb: (seb[b], i, 0)),
            ],
            out_specs=pl.BlockSpec((B, IBLK), lambda i, b, seb: (b, i)),
        ),
        out_shape=jax.ShapeDtypeStruct((NPAD, INTER), jnp.bfloat16),
    )(seb, x_sorted, gate_w, up_w)

    y_sorted = pl.pallas_call(
        _down_body,
        grid_spec=pltpu.PrefetchScalarGridSpec(
            num_scalar_prefetch=1,
            grid=(NBMAX,),
            in_specs=[
                pl.BlockSpec((B, INTER), lambda b, seb: (b, 0)),
                pl.BlockSpec((1, HIDDEN, INTER), lambda b, seb: (seb[b], 0, 0)),
            ],
            out_specs=pl.BlockSpec((B, HIDDEN), lambda b, seb: (b, 0)),
        ),
        out_shape=jax.ShapeDtypeStruct((NPAD, HIDDEN), jnp.float32),
    )(seb, h_sorted, down_w)

    p0 = pos_flat[0::2]
    p1 = pos_flat[1::2]
    w1b = jnp.broadcast_to(w_out[:, 0:1], (T, 16))
    w2b = jnp.broadcast_to(w_out[:, 1:2], (T, 16))
    out = _make_combine()(y_sorted, p0, p1, w1b, w2b)

    return out.reshape(*batch_shape, HIDDEN)


# speedup vs baseline: 1.9396x; 1.0207x over previous
import functools

import jax
import jax.numpy as jnp
from jax import lax
from jax.experimental import pallas as pl
from jax.experimental.pallas import tpu as pltpu
from jax.experimental.pallas import tpu_sc as plsc

HIDDEN = 1024
INTER = 2816
E = 8

T = 4096
P = 2 * T
B = 256
NBMAX = P // B + E
NPAD = NBMAX * B
BTR = 1024

NW = 32
PPW = P // NW
CH = 32
NCH = PPW // CH
TPW = T // NW
CC = 16
NCC = TPW // CC


def _routerdisp_body(x_ref, rw_ref, rb_ref, w_ref, pos_ref, eb_ref,
                     sel0_scr, sel1_scr):
    tb = pl.program_id(0)
    x = x_ref[...]
    logits = jnp.dot(x, rw_ref[...].T, preferred_element_type=jnp.float32)
    logits = logits + rb_ref[...]
    m = jnp.max(logits, axis=-1, keepdims=True)
    ex = jnp.exp(logits - m)
    probs = ex / jnp.sum(ex, axis=-1, keepdims=True)

    lane = lax.broadcasted_iota(jnp.int32, (BTR, E), 1)
    m1 = jnp.max(probs, axis=-1, keepdims=True)
    a1 = jnp.min(jnp.where(probs == m1, lane, E), axis=-1, keepdims=True)
    probs2 = jnp.where(lane == a1, -1.0, probs)
    m2 = jnp.max(probs2, axis=-1, keepdims=True)
    a2 = jnp.min(jnp.where(probs2 == m2, lane, E), axis=-1, keepdims=True)

    denom = m1 + m2
    w1 = m1 / denom
    w2 = m2 / denom
    zf = jnp.zeros((BTR, 126), jnp.float32)
    w_ref[...] = jnp.concatenate([w1, w2, zf], axis=-1)

    rows = BTR // 128
    sel0_scr[pl.ds(tb * rows, rows), :] = jnp.reshape(a1, (rows, 128))
    sel1_scr[pl.ds(tb * rows, rows), :] = jnp.reshape(a2, (rows, 128))

    @pl.when(tb == T // BTR - 1)
    def _dispatch():
        R = jnp.concatenate([sel0_scr[...], sel1_scr[...]], axis=0)
        _dispatch_math(R, pos_ref, eb_ref)


def _dispatch_math(R, pos_ref, eb_ref):
    r0 = lax.broadcasted_iota(jnp.int32, (128, 128), 0)
    r1 = lax.broadcasted_iota(jnp.int32, (128, 128), 1)
    SU = (r0 < r1).astype(jnp.float32)
    s0 = lax.broadcasted_iota(jnp.int32, (64, 64), 0)
    s1 = lax.broadcasted_iota(jnp.int32, (64, 64), 1)
    SL = (s1 < s0).astype(jnp.float32)

    pos = jnp.zeros((64, 128), jnp.int32)
    blk = lax.broadcasted_iota(jnp.int32, (1, 128), 1)
    ebv = jnp.zeros((1, 128), jnp.int32)
    base = jnp.int32(0)
    for e in range(E):
        M = (R == e).astype(jnp.float32)
        lanepre = jnp.dot(M, SU, preferred_element_type=jnp.float32)
        tot = jnp.sum(M, axis=1, keepdims=True)
        rowpre = jnp.dot(SL, tot, preferred_element_type=jnp.float32)
        rank = (lanepre + rowpre).astype(jnp.int32)
        cnt = jnp.sum(M).astype(jnp.int32)
        cntpad = ((cnt + B - 1) // B) * B
        pos = jnp.where(R == e, base + rank, pos)
        base = base + cntpad
        ebv = ebv + (blk * B >= base).astype(jnp.int32)
    pos_ref[...] = pos
    eb_ref[...] = jnp.where(blk == 127, base // B, jnp.minimum(ebv, E - 1))


IBLK = 2816
NI = INTER // IBLK


def _gateup_body(seb_ref, x_ref, gw_ref, uw_ref, h_ref):
    b = pl.program_id(1)
    nact = seb_ref[127]

    @pl.when(b < nact)
    def _():
        x = x_ref[...]
        g = jnp.dot(x, gw_ref[0].T, preferred_element_type=jnp.float32)
        u = jnp.dot(x, uw_ref[0].T, preferred_element_type=jnp.float32)
        h = g * lax.logistic(g) * u
        h_ref[...] = h.astype(jnp.bfloat16)


def _down_body(seb_ref, h_ref, dw_ref, y_ref):
    b = pl.program_id(0)
    nact = seb_ref[127]

    @pl.when(b < nact)
    def _():
        h = h_ref[...].astype(jnp.float32)
        y_ref[...] = jnp.dot(h, dw_ref[0].T,
                             preferred_element_type=jnp.float32)


def _make_gather():
    mesh = plsc.VectorSubcoreMesh(core_axis_name="c", subcore_axis_name="s")

    @functools.partial(
        pl.kernel, mesh=mesh,
        out_type=jax.ShapeDtypeStruct((NPAD, HIDDEN), jnp.float32),
        scratch_types=[
            pltpu.VMEM((PPW,), jnp.int32),
            pltpu.VMEM((PPW,), jnp.int32),
            pltpu.VMEM((CH, HIDDEN), jnp.float32),
            pltpu.VMEM((CH, HIDDEN), jnp.float32),
            pltpu.SemaphoreType.DMA,
            pltpu.SemaphoreType.DMA,
            pltpu.SemaphoreType.DMA,
            pltpu.SemaphoreType.DMA,
        ],
    )
    def gather_k(x_hbm, tok_hbm, pos_hbm, xs_hbm, tok_v, pos_v,
                 rows0, rows1, sg0, sg1, ss0, ss1):
        wid = lax.axis_index("s") * 2 + lax.axis_index("c")
        base = wid * PPW
        pltpu.sync_copy(tok_hbm.at[pl.ds(base, PPW)], tok_v)
        pltpu.sync_copy(pos_hbm.at[pl.ds(base, PPW)], pos_v)
        rows = (rows0, rows1)
        sg = (sg0, sg1)
        ss = (ss0, ss1)

        def g_start(c, s):
            idx = tok_v.at[pl.ds(c * CH, CH)]
            return pltpu.async_copy(x_hbm.at[idx], rows[s], sg[s])

        def s_start(c, s):
            idx = pos_v.at[pl.ds(c * CH, CH)]
            return pltpu.async_copy(rows[s], xs_hbm.at[idx], ss[s])

        gath = [None, None]
        scat = [None, None]
        gath[0] = g_start(0, 0)
        for c in range(NCH):
            s = c % 2
            if c + 1 < NCH:
                if scat[1 - s] is not None:
                    scat[1 - s].wait()
                    scat[1 - s] = None
                gath[1 - s] = g_start(c + 1, 1 - s)
            gath[s].wait()
            scat[s] = s_start(c, s)
        for s in range(2):
            if scat[s] is not None:
                scat[s].wait()

    return gather_k


def _make_combine():
    mesh = plsc.VectorSubcoreMesh(core_axis_name="c", subcore_axis_name="s")

    @functools.partial(
        pl.kernel, mesh=mesh,
        out_type=jax.ShapeDtypeStruct((T, HIDDEN), jnp.float32),
        scratch_types=[
            pltpu.VMEM((TPW,), jnp.int32),
            pltpu.VMEM((TPW,), jnp.int32),
            pltpu.VMEM((TPW, 16), jnp.float32),
            pltpu.VMEM((TPW, 16), jnp.float32),
            pltpu.VMEM((CC, HIDDEN), jnp.float32),
            pltpu.VMEM((CC, HIDDEN), jnp.float32),
            pltpu.VMEM((CC, HIDDEN), jnp.float32),
            pltpu.VMEM((CC, HIDDEN), jnp.float32),
            pltpu.SemaphoreType.DMA,
            pltpu.SemaphoreType.DMA,
            pltpu.SemaphoreType.DMA,
            pltpu.SemaphoreType.DMA,
            pltpu.SemaphoreType.DMA,
            pltpu.SemaphoreType.DMA,
        ],
    )
    def combine_k(y_hbm, p0_hbm, p1_hbm, w1_hbm, w2_hbm, out_hbm,
                  i0_v, i1_v, w1_v, w2_v, y0a, y0b, y1a, y1b,
                  sa0, sb0, sa1, sb1, soa, sob):
        wid = lax.axis_index("s") * 2 + lax.axis_index("c")
        base = wid * TPW
        pltpu.sync_copy(p0_hbm.at[pl.ds(base, TPW)], i0_v)
        pltpu.sync_copy(p1_hbm.at[pl.ds(base, TPW)], i1_v)
        pltpu.sync_copy(w1_hbm.at[pl.ds(base, TPW)], w1_v)
        pltpu.sync_copy(w2_hbm.at[pl.ds(base, TPW)], w2_v)
        y0 = (y0a, y0b)
        y1 = (y1a, y1b)
        s0 = (sa0, sb0)
        s1 = (sa1, sb1)
        so = (soa, sob)

        def g_start(c, s):
            cp0 = pltpu.async_copy(
                y_hbm.at[i0_v.at[pl.ds(c * CC, CC)]], y0[s], s0[s])
            cp1 = pltpu.async_copy(
                y_hbm.at[i1_v.at[pl.ds(c * CC, CC)]], y1[s], s1[s])
            return cp0, cp1

        gath = [None, None]
        stor = [None, None]
        gath[0] = g_start(0, 0)
        for c in range(NCC):
            s = c % 2
            if c + 1 < NCC:
                if stor[1 - s] is not None:
                    stor[1 - s].wait()
                    stor[1 - s] = None
                gath[1 - s] = g_start(c + 1, 1 - s)
            gath[s][0].wait()
            gath[s][1].wait()

            def tok(j, carry2, _s=s, _c=c):
                wv1 = w1_v[_c * CC + j]
                wv2 = w2_v[_c * CC + j]
                for k in range(HIDDEN // 16):
                    sl = pl.ds(k * 16, 16)
                    y0[_s][j, sl] = wv1 * y0[_s][j, sl] + wv2 * y1[_s][j, sl]
                return carry2

            lax.fori_loop(0, CC, tok, 0)
            stor[s] = pltpu.async_copy(
                y0[s], out_hbm.at[pl.ds(base + c * CC, CC)], so[s])
        for s in range(2):
            if stor[s] is not None:
                stor[s].wait()

    return combine_k


@jax.jit
def kernel(x, router_w, router_b, gate_w, up_w, down_w):
    batch_shape = x.shape[:-1]
    xf = x.reshape(-1, HIDDEN)

    w_out, pos, eb = pl.pallas_call(
        _routerdisp_body,
        grid=(T // BTR,),
        in_specs=[
            pl.BlockSpec((BTR, HIDDEN), lambda t: (t, 0)),
            pl.BlockSpec((E, HIDDEN), lambda t: (0, 0)),
            pl.BlockSpec((1, E), lambda t: (0, 0)),
        ],
        out_specs=[
            pl.BlockSpec((BTR, 128), lambda t: (t, 0)),
            pl.BlockSpec((64, 128), lambda t: (0, 0)),
            pl.BlockSpec((1, 128), lambda t: (0, 0)),
        ],
        out_shape=[
            jax.ShapeDtypeStruct((T, 128), jnp.float32),
            jax.ShapeDtypeStruct((64, 128), jnp.int32),
            jax.ShapeDtypeStruct((1, 128), jnp.int32),
        ],
        scratch_shapes=[
            pltpu.VMEM((32, 128), jnp.int32),
            pltpu.VMEM((32, 128), jnp.int32),
        ],
    )(xf, router_w, router_b.reshape(1, E))

    pos_flat = pos.reshape(P)
    tok_flat = jnp.arange(P, dtype=jnp.int32) % T
    x_sorted = _make_gather()(xf, tok_flat, pos_flat)

    seb = eb.reshape(128)
    h_sorted = pl.pallas_call(
        _gateup_body,
        grid_spec=pltpu.PrefetchScalarGridSpec(
            num_scalar_prefetch=1,
            grid=(NI, NBMAX),
            in_specs=[
                pl.BlockSpec((B, HIDDEN), lambda i, b, seb: (b, 0)),
                pl.BlockSpec((1, IBLK, HIDDEN),
                             lambda i, b, seb: (seb[b], i, 0)),
                pl.BlockSpec((1, IBLK, HIDDEN),
                             lambda i, b, seb: (seb[b], i, 0)),
            ],
            out_specs=pl.BlockSpec((B, IBLK), lambda i, b, seb: (b, i)),
        ),
        out_shape=jax.ShapeDtypeStruct((NPAD, INTER), jnp.bfloat16),
    )(seb, x_sorted, gate_w, up_w)

    y_sorted = pl.pallas_call(
        _down_body,
        grid_spec=pltpu.PrefetchScalarGridSpec(
            num_scalar_prefetch=1,
            grid=(NBMAX,),
            in_specs=[
                pl.BlockSpec((B, INTER), lambda b, seb: (b, 0)),
                pl.BlockSpec((1, HIDDEN, INTER), lambda b, seb: (seb[b], 0, 0)),
            ],
            out_specs=pl.BlockSpec((B, HIDDEN), lambda b, seb: (b, 0)),
        ),
        out_shape=jax.ShapeDtypeStruct((NPAD, HIDDEN), jnp.float32),
    )(seb, h_sorted, down_w)

    p0 = pos_flat[:T]
    p1 = pos_flat[T:]
    w1b = jnp.broadcast_to(w_out[:, 0:1], (T, 16))
    w2b = jnp.broadcast_to(w_out[:, 1:2], (T, 16))
    out = _make_combine()(y_sorted, p0, p1, w1b, w2b)

    return out.reshape(*batch_shape, HIDDEN)
